# Initial kernel scaffold; baseline (speedup 1.0000x reference)
import jax
import jax.numpy as jnp
from jax.experimental import pallas as pl


def kernel(id_indices, table, W1, b1, W2, b2):
    raise NotImplementedError("write your pallas kernel here")


# speedup vs baseline: 1.3243x; 1.3243x over previous
import functools

import jax
import jax.numpy as jnp
from jax import lax
from jax.experimental import pallas as pl
from jax.experimental.pallas import tpu as pltpu
from jax.experimental.pallas import tpu_sc as plsc

EMB = 32
HID = 16
TOTAL = 16384 * 26
NC = 2
NS = 16
NW = NC * NS
PER_W = TOTAL // NW
IDXV = 128
GPC = 13
CHUNK = IDXV * GPC
NCHUNK = PER_W // CHUNK
IDX_ROWS = TOTAL // IDXV
IDXR_W = PER_W // IDXV


def _gather_body(table_hbm, idx_hbm, out_hbm, idx_v, rows_v, sem):
    c = lax.axis_index("c")
    s = lax.axis_index("s")
    wid = s * NC + c
    pltpu.sync_copy(idx_hbm.at[pl.ds(wid * IDXR_W, IDXR_W)], idx_v)

    def chunk_body(k, carry):
        base = wid * PER_W + k * CHUNK
        cps = [
            pltpu.async_copy(
                table_hbm.at[idx_v.at[k * GPC + j]],
                rows_v.at[pl.ds(j * IDXV, IDXV)],
                sem,
            )
            for j in range(GPC)
        ]
        for cp in cps:
            cp.wait()
        pltpu.sync_copy(rows_v, out_hbm.at[pl.ds(base, CHUNK)])
        return carry

    lax.fori_loop(0, NCHUNK, chunk_body, 0)


_sc_gather = functools.partial(
    pl.kernel,
    mesh=plsc.VectorSubcoreMesh(core_axis_name="c", subcore_axis_name="s"),
    out_type=jax.ShapeDtypeStruct((TOTAL, EMB), jnp.float32),
    scratch_types=[
        pltpu.VMEM((IDXR_W, IDXV), jnp.int32),
        pltpu.VMEM((CHUNK, EMB), jnp.float32),
        pltpu.SemaphoreType.DMA,
    ],
    compiler_params=pltpu.CompilerParams(use_tc_tiling_on_sc=False),
)(_gather_body)


MLP_BLK = 8192
MLP_GRID = TOTAL // MLP_BLK


def _mlp_body(x_ref, w1_ref, b1_ref, w2_ref, b2_ref, o_ref):
    x = x_ref[...]
    h = jnp.dot(x, w1_ref[...], preferred_element_type=jnp.float32) + b1_ref[...]
    h = jnp.maximum(h, 0.0)
    o_ref[...] = jnp.dot(h, w2_ref[...], preferred_element_type=jnp.float32) + b2_ref[...]


def _mlp(emb2d, W1, b1, W2, b2):
    return pl.pallas_call(
        _mlp_body,
        grid=(MLP_GRID,),
        in_specs=[
            pl.BlockSpec((MLP_BLK, EMB), lambda i: (i, 0)),
            pl.BlockSpec((EMB, HID), lambda i: (0, 0)),
            pl.BlockSpec((1, HID), lambda i: (0, 0)),
            pl.BlockSpec((HID, 1), lambda i: (0, 0)),
            pl.BlockSpec((1, 1), lambda i: (0, 0)),
        ],
        out_specs=pl.BlockSpec((MLP_BLK, 1), lambda i: (i, 0)),
        out_shape=jax.ShapeDtypeStruct((TOTAL, 1), jnp.float32),
    )(emb2d, W1, b1.reshape(1, HID), W2, b2.reshape(1, 1))


def kernel(id_indices, table, W1, b1, W2, b2):
    B, F = id_indices.shape
    idx2d = id_indices.reshape(IDX_ROWS, IDXV).astype(jnp.int32)
    emb2d = _sc_gather(table, idx2d)
    pred = _mlp(emb2d, W1, b1, W2, b2)
    return emb2d.reshape(B, F, EMB), pred.reshape(B, F, 1)


# speedup vs baseline: 1.8720x; 1.4136x over previous
import functools

import jax
import jax.numpy as jnp
from jax import lax
from jax.experimental import pallas as pl
from jax.experimental.pallas import tpu as pltpu
from jax.experimental.pallas import tpu_sc as plsc

EMB = 32
HID = 16
BATCH = 16384
FIELDS = 26
TOTAL = BATCH * FIELDS
NC = 2
NS = 16
NW = NC * NS
PER_W = TOTAL // NW
IDXV = 128
GPC = 13
CHUNK = IDXV * GPC
NCHUNK = PER_W // CHUNK
IDX_ROWS = TOTAL // IDXV
IDXR_W = PER_W // IDXV


def _gather_body(table_hbm, idx_hbm, out_hbm, idx_v, rows_v, sem):
    c = lax.axis_index("c")
    s = lax.axis_index("s")
    wid = s * NC + c
    pltpu.sync_copy(idx_hbm.at[pl.ds(wid * IDXR_W, IDXR_W)], idx_v)

    def chunk_body(k, carry):
        base = wid * PER_W + k * CHUNK
        cps = [
            pltpu.async_copy(
                table_hbm.at[idx_v.at[k * GPC + j]],
                rows_v.at[pl.ds(j * IDXV, IDXV)],
                sem,
            )
            for j in range(GPC)
        ]
        for cp in cps:
            cp.wait()
        pltpu.sync_copy(rows_v, out_hbm.at[pl.ds(base, CHUNK)])
        return carry

    lax.fori_loop(0, NCHUNK, chunk_body, 0)


_sc_gather = functools.partial(
    pl.kernel,
    mesh=plsc.VectorSubcoreMesh(core_axis_name="c", subcore_axis_name="s"),
    out_type=jax.ShapeDtypeStruct((TOTAL, EMB), jnp.float32),
    scratch_types=[
        pltpu.VMEM((IDXR_W, IDXV), jnp.int32),
        pltpu.VMEM((CHUNK, EMB), jnp.float32),
        pltpu.SemaphoreType.DMA,
    ],
    compiler_params=pltpu.CompilerParams(use_tc_tiling_on_sc=False),
)(_gather_body)


def _mlp_body(x_ref, w1t_ref, b1_ref, w2t_ref, b2_ref, embt_ref, predt_ref):
    x = x_ref[...]
    xt = x.T
    embt_ref[...] = xt
    h = jnp.dot(w1t_ref[...], xt, preferred_element_type=jnp.float32)
    h = jnp.maximum(h + b1_ref[...], 0.0)
    p = jnp.dot(w2t_ref[...], h, preferred_element_type=jnp.float32)
    predt_ref[0] = p + b2_ref[...]


def _mlp(emb2d, W1, b1, W2, b2):
    return pl.pallas_call(
        _mlp_body,
        grid=(FIELDS,),
        in_specs=[
            pl.BlockSpec((BATCH, EMB), lambda i: (i, 0)),
            pl.BlockSpec((HID, EMB), lambda i: (0, 0)),
            pl.BlockSpec((HID, 1), lambda i: (0, 0)),
            pl.BlockSpec((1, HID), lambda i: (0, 0)),
            pl.BlockSpec((1, 1), lambda i: (0, 0)),
        ],
        out_specs=[
            pl.BlockSpec((EMB, BATCH), lambda i: (i, 0)),
            pl.BlockSpec((1, 1, BATCH), lambda i: (i, 0, 0)),
        ],
        out_shape=[
            jax.ShapeDtypeStruct((FIELDS * EMB, BATCH), jnp.float32),
            jax.ShapeDtypeStruct((FIELDS, 1, BATCH), jnp.float32),
        ],
    )(emb2d, W1.T, b1.reshape(HID, 1), W2.T, b2.reshape(1, 1))


def kernel(id_indices, table, W1, b1, W2, b2):
    idx2d = id_indices.T.astype(jnp.int32).reshape(IDX_ROWS, IDXV)
    emb2d = _sc_gather(table, idx2d)
    embt, predt = _mlp(emb2d, W1, b1, W2, b2)
    embeddings = embt.reshape(FIELDS, EMB, BATCH).transpose(2, 0, 1)
    prediction = predt.transpose(2, 0, 1)
    return embeddings, prediction


# speedup vs baseline: 3.4000x; 1.8162x over previous
import functools

import jax
import jax.numpy as jnp
from jax import lax
from jax.experimental import pallas as pl
from jax.experimental.pallas import tpu as pltpu
from jax.experimental.pallas import tpu_sc as plsc

EMB = 32
HID = 16
BATCH = 16384
FIELDS = 26
TOTAL = BATCH * FIELDS
NUM_IDS = 1000000
NC = 2
NS = 16
NW = NC * NS
PER_W = TOTAL // NW
IDXV = 128
GPC = 13
CHUNK = IDXV * GPC
NCHUNK = PER_W // CHUNK
IDX_ROWS = TOTAL // IDXV
IDXR_W = PER_W // IDXV

TBLK = 8192
TJ = (NUM_IDS + TBLK - 1) // TBLK
NUM_P = TJ * TBLK
QT = TBLK // 4


def _tr_body(xt_ref, o_ref):
    x = xt_ref[...]
    parts = [x[:, QT * j:QT * (j + 1)].T for j in range(4)]
    o_ref[...] = jnp.concatenate(parts, axis=1)


def _transpose_table(tableT):
    return pl.pallas_call(
        _tr_body,
        grid=(TJ,),
        in_specs=[pl.BlockSpec((EMB, TBLK), lambda j: (0, j))],
        out_specs=pl.BlockSpec((QT, 128), lambda j: (j, 0)),
        out_shape=jax.ShapeDtypeStruct((NUM_P // 4, 128), jnp.float32),
    )(tableT)


def _perm(i):
    m = i & (TBLK - 1)
    c0 = i - m
    return c0 + 4 * (m & (QT - 1)) + (m >> 11)


def _gather_body(table_hbm, idx_hbm, out_hbm, idx_v, rows_v, sem):
    c = lax.axis_index("c")
    s = lax.axis_index("s")
    wid = s * NC + c
    pltpu.sync_copy(idx_hbm.at[pl.ds(wid * IDXR_W, IDXR_W)], idx_v)

    def chunk_body(k, carry):
        base = wid * PER_W + k * CHUNK
        cps = [
            pltpu.async_copy(
                table_hbm.at[idx_v.at[k * GPC + j]],
                rows_v.at[pl.ds(j * IDXV, IDXV)],
                sem,
            )
            for j in range(GPC)
        ]
        for cp in cps:
            cp.wait()
        pltpu.sync_copy(rows_v, out_hbm.at[pl.ds(base, CHUNK)])
        return carry

    lax.fori_loop(0, NCHUNK, chunk_body, 0)


_sc_gather = functools.partial(
    pl.kernel,
    mesh=plsc.VectorSubcoreMesh(core_axis_name="c", subcore_axis_name="s"),
    out_type=jax.ShapeDtypeStruct((TOTAL, EMB), jnp.float32),
    scratch_types=[
        pltpu.VMEM((IDXR_W, IDXV), jnp.int32),
        pltpu.VMEM((CHUNK, EMB), jnp.float32),
        pltpu.SemaphoreType.DMA,
    ],
    compiler_params=pltpu.CompilerParams(use_tc_tiling_on_sc=False),
)(_gather_body)


QB = BATCH // 4


def _mlp_body(xw_ref, w1t_ref, b1_ref, w2t_ref, b2_ref, embt_ref, predt_ref):
    xw = xw_ref[...]
    parts = [xw[:, EMB * j:EMB * (j + 1)].T for j in range(4)]
    xt = jnp.concatenate(parts, axis=1)
    embt_ref[...] = xt
    h = jnp.dot(w1t_ref[...], xt, preferred_element_type=jnp.float32)
    h = jnp.maximum(h + b1_ref[...], 0.0)
    p = jnp.dot(w2t_ref[...], h, preferred_element_type=jnp.float32)
    predt_ref[0] = p + b2_ref[...]


def _mlp(emb_c, W1, b1, W2, b2):
    return pl.pallas_call(
        _mlp_body,
        grid=(FIELDS,),
        in_specs=[
            pl.BlockSpec((QB, 128), lambda i: (i, 0)),
            pl.BlockSpec((HID, EMB), lambda i: (0, 0)),
            pl.BlockSpec((HID, 1), lambda i: (0, 0)),
            pl.BlockSpec((1, HID), lambda i: (0, 0)),
            pl.BlockSpec((1, 1), lambda i: (0, 0)),
        ],
        out_specs=[
            pl.BlockSpec((EMB, BATCH), lambda i: (i, 0)),
            pl.BlockSpec((1, 1, BATCH), lambda i: (i, 0, 0)),
        ],
        out_shape=[
            jax.ShapeDtypeStruct((FIELDS * EMB, BATCH), jnp.float32),
            jax.ShapeDtypeStruct((FIELDS, 1, BATCH), jnp.float32),
        ],
    )(emb_c, W1.T, b1.reshape(HID, 1), W2.T, b2.reshape(1, 1))


def kernel(id_indices, table, W1, b1, W2, b2):
    idx_r = (
        id_indices.T.astype(jnp.int32)
        .reshape(FIELDS, 4, IDXV, EMB)
        .transpose(0, 2, 3, 1)
        .reshape(IDX_ROWS, IDXV)
    )
    idx2d = _perm(idx_r)
    table_c = _transpose_table(table.T).reshape(NUM_P, EMB)
    emb2d = _sc_gather(table_c, idx2d)
    emb_c = emb2d.reshape(TOTAL // 4, 128)
    embt, predt = _mlp(emb_c, W1, b1, W2, b2)
    embeddings = embt.reshape(FIELDS, EMB, BATCH).transpose(2, 0, 1)
    prediction = predt.transpose(2, 0, 1)
    return embeddings, prediction


# speedup vs baseline: 5.7076x; 1.6787x over previous
import functools

import jax
import jax.numpy as jnp
from jax import lax
from jax.experimental import pallas as pl
from jax.experimental.pallas import tpu as pltpu
from jax.experimental.pallas import tpu_sc as plsc

EMB = 32
HID = 16
BATCH = 16384
FIELDS = 26
TOTAL = BATCH * FIELDS
NUM_IDS = 1000000
NC = 2
NS = 16
NW = NC * NS
PER_W = TOTAL // NW
IDXV = 128
GPC = 13
CHUNK = IDXV * GPC
NCHUNK = PER_W // CHUNK
IDX_ROWS = TOTAL // IDXV
IDXR_W = PER_W // IDXV

TBLK = 8192
TJ = (NUM_IDS + TBLK - 1) // TBLK
NUM_P = TJ * TBLK
QT = TBLK // 4


def _tr_body(xt_ref, o_ref):
    x = xt_ref[...]
    u = jnp.concatenate([x[:, QT * j:QT * (j + 1)] for j in range(4)], axis=0)
    o_ref[...] = u.T


def _transpose_table(tableT):
    return pl.pallas_call(
        _tr_body,
        grid=(TJ,),
        in_specs=[pl.BlockSpec((EMB, TBLK), lambda j: (0, j))],
        out_specs=pl.BlockSpec((QT, 128), lambda j: (j, 0)),
        out_shape=jax.ShapeDtypeStruct((NUM_P // 4, 128), jnp.float32),
    )(tableT)


def _perm(i):
    m = i & (TBLK - 1)
    c0 = i - m
    return c0 + 4 * (m & (QT - 1)) + (m >> 11)


def _gather_body(table_hbm, idx_hbm, out_hbm, idx_v, rows_v, sem):
    c = lax.axis_index("c")
    s = lax.axis_index("s")
    wid = s * NC + c
    pltpu.sync_copy(idx_hbm.at[pl.ds(wid * IDXR_W, IDXR_W)], idx_v)

    def chunk_body(k, carry):
        base = wid * PER_W + k * CHUNK
        cps = [
            pltpu.async_copy(
                table_hbm.at[idx_v.at[k * GPC + j]],
                rows_v.at[pl.ds(j * IDXV, IDXV)],
                sem,
            )
            for j in range(GPC)
        ]
        for cp in cps:
            cp.wait()
        pltpu.sync_copy(rows_v, out_hbm.at[pl.ds(base, CHUNK)])
        return carry

    lax.fori_loop(0, NCHUNK, chunk_body, 0)


_sc_gather = functools.partial(
    pl.kernel,
    mesh=plsc.VectorSubcoreMesh(core_axis_name="c", subcore_axis_name="s"),
    out_type=jax.ShapeDtypeStruct((TOTAL, EMB), jnp.float32),
    scratch_types=[
        pltpu.VMEM((IDXR_W, IDXV), jnp.int32),
        pltpu.VMEM((CHUNK, EMB), jnp.float32),
        pltpu.SemaphoreType.DMA,
    ],
    compiler_params=pltpu.CompilerParams(use_tc_tiling_on_sc=False),
)(_gather_body)


QB = BATCH // 4


def _mlp_body(xw_ref, w1t_ref, b1_ref, w2t_ref, b2_ref, embt_ref, predt_ref):
    xw = xw_ref[...]
    xwt = xw.T
    xt = jnp.concatenate(
        [xwt[EMB * j:EMB * (j + 1), :] for j in range(4)], axis=1
    )
    embt_ref[...] = xt
    h = jnp.dot(w1t_ref[...], xt, preferred_element_type=jnp.float32)
    h = jnp.maximum(h + b1_ref[...], 0.0)
    p = jnp.dot(w2t_ref[...], h, preferred_element_type=jnp.float32)
    predt_ref[0] = p + b2_ref[...]


def _mlp(emb_c, W1, b1, W2, b2):
    return pl.pallas_call(
        _mlp_body,
        grid=(FIELDS,),
        in_specs=[
            pl.BlockSpec((QB, 128), lambda i: (i, 0)),
            pl.BlockSpec((HID, EMB), lambda i: (0, 0)),
            pl.BlockSpec((HID, 1), lambda i: (0, 0)),
            pl.BlockSpec((1, HID), lambda i: (0, 0)),
            pl.BlockSpec((1, 1), lambda i: (0, 0)),
        ],
        out_specs=[
            pl.BlockSpec((EMB, BATCH), lambda i: (i, 0)),
            pl.BlockSpec((1, 1, BATCH), lambda i: (i, 0, 0)),
        ],
        out_shape=[
            jax.ShapeDtypeStruct((FIELDS * EMB, BATCH), jnp.float32),
            jax.ShapeDtypeStruct((FIELDS, 1, BATCH), jnp.float32),
        ],
    )(emb_c, W1.T, b1.reshape(HID, 1), W2.T, b2.reshape(1, 1))


def kernel(id_indices, table, W1, b1, W2, b2):
    idx_r = (
        id_indices.T.astype(jnp.int32)
        .reshape(FIELDS, 4, IDXV, EMB)
        .transpose(0, 2, 3, 1)
        .reshape(IDX_ROWS, IDXV)
    )
    idx2d = _perm(idx_r)
    table_c = _transpose_table(table.T).reshape(NUM_P, EMB)
    emb2d = _sc_gather(table_c, idx2d)
    emb_c = emb2d.reshape(TOTAL // 4, 128)
    embt, predt = _mlp(emb_c, W1, b1, W2, b2)
    embeddings = embt.reshape(FIELDS, EMB, BATCH).transpose(2, 0, 1)
    prediction = predt.transpose(2, 0, 1)
    return embeddings, prediction


# speedup vs baseline: 7.3210x; 1.2827x over previous
import functools

import jax
import jax.numpy as jnp
from jax import lax
from jax.experimental import pallas as pl
from jax.experimental.pallas import tpu as pltpu
from jax.experimental.pallas import tpu_sc as plsc

EMB = 32
HID = 16
BATCH = 16384
FIELDS = 26
TOTAL = BATCH * FIELDS
NUM_IDS = 1000000
NC = 2
NS = 16
NW = NC * NS
PER_W = TOTAL // NW
IDXV = 128
GPC = 8
CHUNK = IDXV * GPC
NCHUNK = PER_W // CHUNK
IDX_ROWS = TOTAL // IDXV
IDXR_W = PER_W // IDXV

TBLK = 32768
TJ = (NUM_IDS + TBLK - 1) // TBLK
NUM_P = TJ * TBLK
QT = TBLK // 4
QT_LOG = QT.bit_length() - 1


def _tr_body(xt_ref, o_ref):
    x = xt_ref[...]
    u = jnp.concatenate([x[:, QT * j:QT * (j + 1)] for j in range(4)], axis=0)
    o_ref[...] = u.T


def _transpose_table(tableT):
    return pl.pallas_call(
        _tr_body,
        grid=(TJ,),
        in_specs=[pl.BlockSpec((EMB, TBLK), lambda j: (0, j))],
        out_specs=pl.BlockSpec((QT, 128), lambda j: (j, 0)),
        out_shape=jax.ShapeDtypeStruct((NUM_P // 4, 128), jnp.float32),
    )(tableT)


def _perm(i):
    m = i & (TBLK - 1)
    c0 = i - m
    return c0 + 4 * (m & (QT - 1)) + (m >> QT_LOG)


def _gather_body(table_hbm, idx_hbm, out_hbm, idx_v, rows0, rows1, sem0, sem1):
    c = lax.axis_index("c")
    s = lax.axis_index("s")
    wid = s * NC + c
    base0 = wid * PER_W
    pltpu.sync_copy(idx_hbm.at[pl.ds(wid * IDXR_W, IDXR_W)], idx_v)

    def fire(k, buf, sem):
        for j in range(GPC):
            pltpu.async_copy(
                table_hbm.at[idx_v.at[k * GPC + j]],
                buf.at[pl.ds(j * IDXV, IDXV)],
                sem,
            )

    def drain_write(k, buf, sem):
        pltpu.make_async_copy(out_hbm.at[pl.ds(0, CHUNK)], buf, sem).wait()
        pltpu.sync_copy(buf, out_hbm.at[pl.ds(base0 + k * CHUNK, CHUNK)])

    fire(0, rows0, sem0)

    def pair_body(t, carry):
        k = t * 2
        fire(k + 1, rows1, sem1)
        drain_write(k, rows0, sem0)
        fire(k + 2, rows0, sem0)
        drain_write(k + 1, rows1, sem1)
        return carry

    lax.fori_loop(0, (NCHUNK - 1) // 2, pair_body, 0)
    drain_write(NCHUNK - 1, rows0, sem0)


_sc_gather = functools.partial(
    pl.kernel,
    mesh=plsc.VectorSubcoreMesh(core_axis_name="c", subcore_axis_name="s"),
    out_type=jax.ShapeDtypeStruct((TOTAL, EMB), jnp.float32),
    scratch_types=[
        pltpu.VMEM((IDXR_W, IDXV), jnp.int32),
        pltpu.VMEM((CHUNK, EMB), jnp.float32),
        pltpu.VMEM((CHUNK, EMB), jnp.float32),
        pltpu.SemaphoreType.DMA,
        pltpu.SemaphoreType.DMA,
    ],
    compiler_params=pltpu.CompilerParams(use_tc_tiling_on_sc=False),
)(_gather_body)


QB = BATCH // 4


def _mlp_body(xw_ref, w1t_ref, b1_ref, w2t_ref, b2_ref, embt_ref, predt_ref):
    xw = xw_ref[...]
    xwt = xw.T
    xt = jnp.concatenate(
        [xwt[EMB * j:EMB * (j + 1), :] for j in range(4)], axis=1
    )
    embt_ref[...] = xt
    h = jnp.dot(w1t_ref[...], xt, preferred_element_type=jnp.float32)
    h = jnp.maximum(h + b1_ref[...], 0.0)
    p = jnp.dot(w2t_ref[...], h, preferred_element_type=jnp.float32)
    predt_ref[0] = p + b2_ref[...]


def _mlp(emb_c, W1, b1, W2, b2):
    return pl.pallas_call(
        _mlp_body,
        grid=(FIELDS,),
        in_specs=[
            pl.BlockSpec((QB, 128), lambda i: (i, 0)),
            pl.BlockSpec((HID, EMB), lambda i: (0, 0)),
            pl.BlockSpec((HID, 1), lambda i: (0, 0)),
            pl.BlockSpec((1, HID), lambda i: (0, 0)),
            pl.BlockSpec((1, 1), lambda i: (0, 0)),
        ],
        out_specs=[
            pl.BlockSpec((EMB, BATCH), lambda i: (i, 0)),
            pl.BlockSpec((1, 1, BATCH), lambda i: (i, 0, 0)),
        ],
        out_shape=[
            jax.ShapeDtypeStruct((FIELDS * EMB, BATCH), jnp.float32),
            jax.ShapeDtypeStruct((FIELDS, 1, BATCH), jnp.float32),
        ],
    )(emb_c, W1.T, b1.reshape(HID, 1), W2.T, b2.reshape(1, 1))


def kernel(id_indices, table, W1, b1, W2, b2):
    idx_r = (
        id_indices.T.astype(jnp.int32)
        .reshape(FIELDS, 4, IDXV, EMB)
        .transpose(0, 2, 3, 1)
        .reshape(IDX_ROWS, IDXV)
    )
    idx2d = _perm(idx_r)
    table_c = _transpose_table(table.T).reshape(NUM_P, EMB)
    emb2d = _sc_gather(table_c, idx2d)
    emb_c = emb2d.reshape(TOTAL // 4, 128)
    embt, predt = _mlp(emb_c, W1, b1, W2, b2)
    embeddings = embt.reshape(FIELDS, EMB, BATCH).transpose(2, 0, 1)
    prediction = predt.transpose(2, 0, 1)
    return embeddings, prediction


# speedup vs baseline: 8.7919x; 1.2009x over previous
import functools

import jax
import jax.numpy as jnp
import numpy as np
from jax import lax
from jax.experimental import pallas as pl
from jax.experimental.pallas import tpu as pltpu
from jax.experimental.pallas import tpu_sc as plsc

EMB = 32
HID = 16
WPR = EMB // 2
BATCH = 16384
FIELDS = 26
TOTAL = BATCH * FIELDS
NUM_IDS = 1000000
NC = 2
NS = 16
NW = NC * NS
PER_W = TOTAL // NW
IDXV = 128
GPC = 8
CHUNK = IDXV * GPC
NCHUNK = PER_W // CHUNK
IDX_ROWS = TOTAL // IDXV
IDXR_W = PER_W // IDXV

TBLK = 32768
TJ = (NUM_IDS + TBLK - 1) // TBLK
NUM_P = TJ * TBLK
QT = TBLK // 8

_HI = np.uint32(0xFFFF0000)
_RND = np.uint32(0x8000)


def _pack16(a, b):
    au = lax.bitcast_convert_type(a, jnp.uint32) + _RND
    bu = lax.bitcast_convert_type(b, jnp.uint32) + _RND
    return lax.bitcast_convert_type(
        (au & _HI) | lax.shift_right_logical(bu, np.uint32(16)), jnp.int32
    )


def _tr_body(xt_ref, o_ref):
    x = xt_ref[...]
    parts = [
        _pack16(x[0:WPR, QT * j:QT * (j + 1)], x[WPR:EMB, QT * j:QT * (j + 1)])
        for j in range(8)
    ]
    u = jnp.concatenate(parts, axis=0)
    o_ref[...] = u.T


def _pack_table(tableT):
    return pl.pallas_call(
        _tr_body,
        grid=(TJ,),
        in_specs=[pl.BlockSpec((EMB, TBLK), lambda j: (0, j))],
        out_specs=pl.BlockSpec((QT, 128), lambda j: (j, 0)),
        out_shape=jax.ShapeDtypeStruct((NUM_P // 8, 128), jnp.int32),
    )(tableT)


def _perm(i):
    m = i & (TBLK - 1)
    c0 = i - m
    return c0 + 8 * (m & (QT - 1)) + (m >> 12)


def _gather_body(table_hbm, idx_hbm, out_hbm, idx_v, rows0, rows1, sem0, sem1):
    c = lax.axis_index("c")
    s = lax.axis_index("s")
    wid = s * NC + c
    base0 = wid * PER_W
    pltpu.sync_copy(idx_hbm.at[pl.ds(wid * IDXR_W, IDXR_W)], idx_v)

    def fire(k, buf, sem):
        for j in range(GPC):
            pltpu.async_copy(
                table_hbm.at[idx_v.at[k * GPC + j]],
                buf.at[pl.ds(j * IDXV, IDXV)],
                sem,
            )

    def drain_write(k, buf, sem):
        pltpu.make_async_copy(out_hbm.at[pl.ds(0, CHUNK)], buf, sem).wait()
        pltpu.sync_copy(buf, out_hbm.at[pl.ds(base0 + k * CHUNK, CHUNK)])

    fire(0, rows0, sem0)

    def pair_body(t, carry):
        k = t * 2
        fire(k + 1, rows1, sem1)
        drain_write(k, rows0, sem0)
        fire(k + 2, rows0, sem0)
        drain_write(k + 1, rows1, sem1)
        return carry

    lax.fori_loop(0, (NCHUNK - 1) // 2, pair_body, 0)
    drain_write(NCHUNK - 1, rows0, sem0)


_sc_gather = functools.partial(
    pl.kernel,
    mesh=plsc.VectorSubcoreMesh(core_axis_name="c", subcore_axis_name="s"),
    out_type=jax.ShapeDtypeStruct((TOTAL, WPR), jnp.int32),
    scratch_types=[
        pltpu.VMEM((IDXR_W, IDXV), jnp.int32),
        pltpu.VMEM((CHUNK, WPR), jnp.int32),
        pltpu.VMEM((CHUNK, WPR), jnp.int32),
        pltpu.SemaphoreType.DMA,
        pltpu.SemaphoreType.DMA,
    ],
    compiler_params=pltpu.CompilerParams(use_tc_tiling_on_sc=False),
)(_gather_body)


QB = BATCH // 8


def _mlp_body(xw_ref, w1t_ref, b1_ref, w2t_ref, b2_ref, embt_ref, predt_ref):
    xw = xw_ref[...]
    xwt = lax.bitcast_convert_type(xw.T, jnp.uint32)
    hi = lax.bitcast_convert_type(xwt & _HI, jnp.float32)
    lo = lax.bitcast_convert_type(
        lax.shift_left(xwt, np.uint32(16)), jnp.float32)
    top = jnp.concatenate([hi[WPR * j:WPR * (j + 1), :] for j in range(8)],
                          axis=1)
    bot = jnp.concatenate([lo[WPR * j:WPR * (j + 1), :] for j in range(8)],
                          axis=1)
    xt = jnp.concatenate([top, bot], axis=0)
    embt_ref[...] = xt
    h = jnp.dot(w1t_ref[...], xt, preferred_element_type=jnp.float32)
    h = jnp.maximum(h + b1_ref[...], 0.0)
    p = jnp.dot(w2t_ref[...], h, preferred_element_type=jnp.float32)
    predt_ref[0] = p + b2_ref[...]


def _mlp(emb_c, W1, b1, W2, b2):
    return pl.pallas_call(
        _mlp_body,
        grid=(FIELDS,),
        in_specs=[
            pl.BlockSpec((QB, 128), lambda i: (i, 0)),
            pl.BlockSpec((HID, EMB), lambda i: (0, 0)),
            pl.BlockSpec((HID, 1), lambda i: (0, 0)),
            pl.BlockSpec((1, HID), lambda i: (0, 0)),
            pl.BlockSpec((1, 1), lambda i: (0, 0)),
        ],
        out_specs=[
            pl.BlockSpec((EMB, BATCH), lambda i: (i, 0)),
            pl.BlockSpec((1, 1, BATCH), lambda i: (i, 0, 0)),
        ],
        out_shape=[
            jax.ShapeDtypeStruct((FIELDS * EMB, BATCH), jnp.float32),
            jax.ShapeDtypeStruct((FIELDS, 1, BATCH), jnp.float32),
        ],
    )(emb_c, W1.T, b1.reshape(HID, 1), W2.T, b2.reshape(1, 1))


def kernel(id_indices, table, W1, b1, W2, b2):
    idx_r = (
        id_indices.T.astype(jnp.int32)
        .reshape(FIELDS, 8, IDXV, WPR)
        .transpose(0, 2, 3, 1)
        .reshape(IDX_ROWS, IDXV)
    )
    idx2d = _perm(idx_r)
    table_c = _pack_table(table.T).reshape(NUM_P, WPR)
    emb2d = _sc_gather(table_c, idx2d)
    emb_c = emb2d.reshape(TOTAL // 8, 128)
    embt, predt = _mlp(emb_c, W1, b1, W2, b2)
    embeddings = embt.reshape(FIELDS, EMB, BATCH).transpose(2, 0, 1)
    prediction = predt.transpose(2, 0, 1)
    return embeddings, prediction


# speedup vs baseline: 9.2692x; 1.0543x over previous
import functools

import jax
import jax.numpy as jnp
import numpy as np
from jax import lax
from jax.experimental import pallas as pl
from jax.experimental.pallas import tpu as pltpu
from jax.experimental.pallas import tpu_sc as plsc

EMB = 32
HID = 16
WPR = EMB // 2
BATCH = 16384
FIELDS = 26
TOTAL = BATCH * FIELDS
NUM_IDS = 1000000
NC = 2
NS = 16
NW = NC * NS
PER_W = TOTAL // NW
IDXV = 128
GPC = 8
CHUNK = IDXV * GPC
NCHUNK = PER_W // CHUNK
IDX_ROWS = TOTAL // IDXV
IDXR_W = PER_W // IDXV

TBLK = 32768
TJ = (NUM_IDS + TBLK - 1) // TBLK
NUM_P = TJ * TBLK
QT = TBLK // 8

_HI = np.uint32(0xFFFF0000)
_RND = np.uint32(0x8000)


def _pack16(a, b):
    au = lax.bitcast_convert_type(a, jnp.uint32) + _RND
    bu = lax.bitcast_convert_type(b, jnp.uint32) + _RND
    return lax.bitcast_convert_type(
        (au & _HI) | lax.shift_right_logical(bu, np.uint32(16)), jnp.int32
    )


def _tr_body(xt_ref, o_ref):
    x = xt_ref[...]
    parts = [
        _pack16(x[0:WPR, QT * j:QT * (j + 1)], x[WPR:EMB, QT * j:QT * (j + 1)])
        for j in range(8)
    ]
    u = jnp.concatenate(parts, axis=0)
    o_ref[...] = u.T


def _pack_table(tableT):
    return pl.pallas_call(
        _tr_body,
        grid=(TJ,),
        in_specs=[pl.BlockSpec((EMB, TBLK), lambda j: (0, j))],
        out_specs=pl.BlockSpec((QT, 128), lambda j: (j, 0)),
        out_shape=jax.ShapeDtypeStruct((NUM_P // 8, 128), jnp.int32),
    )(tableT)


def _perm(i):
    m = i & (TBLK - 1)
    c0 = i - m
    return c0 + 8 * (m & (QT - 1)) + (m >> 12)


def _gather_body(table_hbm, idx_hbm, out_hbm, idx_v, rows0, rows1, sem0, sem1):
    c = lax.axis_index("c")
    s = lax.axis_index("s")
    wid = s * NC + c
    base0 = wid * PER_W
    pltpu.sync_copy(idx_hbm.at[pl.ds(wid * IDXR_W, IDXR_W)], idx_v)

    def fire(k, buf, sem):
        for j in range(GPC):
            pltpu.async_copy(
                table_hbm.at[idx_v.at[k * GPC + j]],
                buf.at[pl.ds(j * IDXV, IDXV)],
                sem,
            )

    def drain_write(k, buf, sem):
        pltpu.make_async_copy(out_hbm.at[pl.ds(0, CHUNK)], buf, sem).wait()
        pltpu.sync_copy(buf, out_hbm.at[pl.ds(base0 + k * CHUNK, CHUNK)])

    fire(0, rows0, sem0)

    def pair_body(t, carry):
        k = t * 2
        fire(k + 1, rows1, sem1)
        drain_write(k, rows0, sem0)
        fire(k + 2, rows0, sem0)
        drain_write(k + 1, rows1, sem1)
        return carry

    lax.fori_loop(0, (NCHUNK - 1) // 2, pair_body, 0)
    drain_write(NCHUNK - 1, rows0, sem0)


_sc_gather = functools.partial(
    pl.kernel,
    mesh=plsc.VectorSubcoreMesh(core_axis_name="c", subcore_axis_name="s"),
    out_type=jax.ShapeDtypeStruct((TOTAL, WPR), jnp.int32),
    scratch_types=[
        pltpu.VMEM((IDXR_W, IDXV), jnp.int32),
        pltpu.VMEM((CHUNK, WPR), jnp.int32),
        pltpu.VMEM((CHUNK, WPR), jnp.int32),
        pltpu.SemaphoreType.DMA,
        pltpu.SemaphoreType.DMA,
    ],
    compiler_params=pltpu.CompilerParams(use_tc_tiling_on_sc=False),
)(_gather_body)


QB = BATCH // 8


FPB = 2


def _unpack_field(xwt_u32):
    hi = lax.bitcast_convert_type(xwt_u32 & _HI, jnp.float32)
    lo = lax.bitcast_convert_type(
        lax.shift_left(xwt_u32, np.uint32(16)), jnp.float32)
    top = jnp.concatenate([hi[WPR * j:WPR * (j + 1), :] for j in range(8)],
                          axis=1)
    bot = jnp.concatenate([lo[WPR * j:WPR * (j + 1), :] for j in range(8)],
                          axis=1)
    return jnp.concatenate([top, bot], axis=0)


def _mlp_body(xw_ref, w1t_ref, b1_ref, w2t_ref, b2_ref, embt_ref, predt_ref):
    xw = xw_ref[...]
    xwt = lax.bitcast_convert_type(xw.T, jnp.uint32)
    for f in range(FPB):
        xt = _unpack_field(xwt[:, QB * f:QB * (f + 1)])
        embt_ref[pl.ds(EMB * f, EMB), :] = xt
        h = jnp.dot(w1t_ref[...], xt, preferred_element_type=jnp.float32)
        h = jnp.maximum(h + b1_ref[...], 0.0)
        p = jnp.dot(w2t_ref[...], h, preferred_element_type=jnp.float32)
        predt_ref[f] = p + b2_ref[...]


def _mlp(emb_c, W1, b1, W2, b2):
    return pl.pallas_call(
        _mlp_body,
        grid=(FIELDS // FPB,),
        in_specs=[
            pl.BlockSpec((FPB * QB, 128), lambda i: (i, 0)),
            pl.BlockSpec((HID, EMB), lambda i: (0, 0)),
            pl.BlockSpec((HID, 1), lambda i: (0, 0)),
            pl.BlockSpec((1, HID), lambda i: (0, 0)),
            pl.BlockSpec((1, 1), lambda i: (0, 0)),
        ],
        out_specs=[
            pl.BlockSpec((FPB * EMB, BATCH), lambda i: (i, 0)),
            pl.BlockSpec((FPB, 1, BATCH), lambda i: (i, 0, 0)),
        ],
        out_shape=[
            jax.ShapeDtypeStruct((FIELDS * EMB, BATCH), jnp.float32),
            jax.ShapeDtypeStruct((FIELDS, 1, BATCH), jnp.float32),
        ],
    )(emb_c, W1.T, b1.reshape(HID, 1), W2.T, b2.reshape(1, 1))


def kernel(id_indices, table, W1, b1, W2, b2):
    idx_r = (
        id_indices.T.astype(jnp.int32)
        .reshape(FIELDS, 8, IDXV, WPR)
        .transpose(0, 2, 3, 1)
        .reshape(IDX_ROWS, IDXV)
    )
    idx2d = _perm(idx_r)
    table_c = _pack_table(table.T).reshape(NUM_P, WPR)
    emb2d = _sc_gather(table_c, idx2d)
    emb_c = emb2d.reshape(TOTAL // 8, 128)
    embt, predt = _mlp(emb_c, W1, b1, W2, b2)
    embeddings = embt.reshape(FIELDS, EMB, BATCH).transpose(2, 0, 1)
    prediction = predt.transpose(2, 0, 1)
    return embeddings, prediction


# speedup vs baseline: 10.2461x; 1.1054x over previous
import functools

import jax
import jax.numpy as jnp
import numpy as np
from jax import lax
from jax.experimental import pallas as pl
from jax.experimental.pallas import tpu as pltpu
from jax.experimental.pallas import tpu_sc as plsc

EMB = 32
HID = 16
WPR = EMB // 2
BATCH = 16384
FIELDS = 26
TOTAL = BATCH * FIELDS
NUM_IDS = 1000000
NC = 2
NS = 16
NW = NC * NS
PER_W = TOTAL // NW
IDXV = 128
GPC = 8
CHUNK = IDXV * GPC
NCHUNK = PER_W // CHUNK
IDX_ROWS = TOTAL // IDXV
IDXR_W = PER_W // IDXV

TBLK = 32768
TJ = (NUM_IDS + TBLK - 1) // TBLK
NUM_P = TJ * TBLK
QT = TBLK // 8

_HI = np.uint32(0xFFFF0000)
_RND = np.uint32(0x8000)


def _pack16(a, b):
    au = lax.bitcast_convert_type(a, jnp.uint32) + _RND
    bu = lax.bitcast_convert_type(b, jnp.uint32) + _RND
    return lax.bitcast_convert_type(
        (au & _HI) | lax.shift_right_logical(bu, np.uint32(16)), jnp.int32
    )


def _tr_body(xt_ref, o_ref):
    x = xt_ref[...]
    parts = [
        _pack16(x[0:WPR, QT * j:QT * (j + 1)], x[WPR:EMB, QT * j:QT * (j + 1)])
        for j in range(8)
    ]
    u = jnp.concatenate(parts, axis=0)
    o_ref[...] = u.T


def _pack_table(tableT):
    return pl.pallas_call(
        _tr_body,
        grid=(TJ,),
        in_specs=[pl.BlockSpec((EMB, TBLK), lambda j: (0, j))],
        out_specs=pl.BlockSpec((QT, 128), lambda j: (j, 0)),
        out_shape=jax.ShapeDtypeStruct((NUM_P // 8, 128), jnp.int32),
    )(tableT)


def _perm(i):
    m = i & (TBLK - 1)
    c0 = i - m
    return c0 + 8 * (m & (QT - 1)) + (m >> 12)


def _shuf_body(idxT_hbm, out_hbm, buf, out_v):
    c = lax.axis_index("c")
    s = lax.axis_index("s")
    wid = s * NC + c
    row0 = wid * IDXR_W
    f0 = jnp.minimum(row0 >> 7, FIELDS - 2)
    pltpu.sync_copy(idxT_hbm.at[pl.ds(f0, 2)], buf)
    lane = lax.broadcasted_iota(jnp.int32, (16,), 0)
    base = (lane & 7) * (BATCH // 8) + (lane >> 3)

    def row_body(t, carry):
        rg = row0 + t
        d = (rg >> 7) - f0
        r = rg & 127
        drow = jnp.full((16,), 0, jnp.int32) + d
        for k in range(8):
            v = plsc.load_gather(buf, [drow, base + (16 * r + 2 * k)])
            m = v & (TBLK - 1)
            pv = v - m + 8 * (m & (QT - 1)) + (m >> 12)
            out_v[pl.ds(t * IDXV + 16 * k, 16)] = pv
        return carry

    lax.fori_loop(0, IDXR_W, row_body, 0)
    pltpu.sync_copy(out_v, out_hbm.at[pl.ds(wid * PER_W, PER_W)])


_sc_shuffle = functools.partial(
    pl.kernel,
    mesh=plsc.VectorSubcoreMesh(core_axis_name="c", subcore_axis_name="s"),
    out_type=jax.ShapeDtypeStruct((TOTAL,), jnp.int32),
    scratch_types=[
        pltpu.VMEM((2, BATCH), jnp.int32),
        pltpu.VMEM((PER_W,), jnp.int32),
    ],
    compiler_params=pltpu.CompilerParams(
        use_tc_tiling_on_sc=False, needs_layout_passes=False
    ),
)(_shuf_body)


def _gather_body(table_hbm, idx_hbm, out_hbm, idx_v, rows0, rows1, sem0, sem1):
    c = lax.axis_index("c")
    s = lax.axis_index("s")
    wid = s * NC + c
    base0 = wid * PER_W
    pltpu.sync_copy(idx_hbm.at[pl.ds(wid * IDXR_W, IDXR_W)], idx_v)

    def fire(k, buf, sem):
        for j in range(GPC):
            pltpu.async_copy(
                table_hbm.at[idx_v.at[k * GPC + j]],
                buf.at[pl.ds(j * IDXV, IDXV)],
                sem,
            )

    def drain_write(k, buf, sem):
        pltpu.make_async_copy(out_hbm.at[pl.ds(0, CHUNK)], buf, sem).wait()
        pltpu.sync_copy(buf, out_hbm.at[pl.ds(base0 + k * CHUNK, CHUNK)])

    fire(0, rows0, sem0)

    def pair_body(t, carry):
        k = t * 2
        fire(k + 1, rows1, sem1)
        drain_write(k, rows0, sem0)
        fire(k + 2, rows0, sem0)
        drain_write(k + 1, rows1, sem1)
        return carry

    lax.fori_loop(0, (NCHUNK - 1) // 2, pair_body, 0)
    drain_write(NCHUNK - 1, rows0, sem0)


_sc_gather = functools.partial(
    pl.kernel,
    mesh=plsc.VectorSubcoreMesh(core_axis_name="c", subcore_axis_name="s"),
    out_type=jax.ShapeDtypeStruct((TOTAL, WPR), jnp.int32),
    scratch_types=[
        pltpu.VMEM((IDXR_W, IDXV), jnp.int32),
        pltpu.VMEM((CHUNK, WPR), jnp.int32),
        pltpu.VMEM((CHUNK, WPR), jnp.int32),
        pltpu.SemaphoreType.DMA,
        pltpu.SemaphoreType.DMA,
    ],
    compiler_params=pltpu.CompilerParams(use_tc_tiling_on_sc=False),
)(_gather_body)


QB = BATCH // 8


FPB = 2


def _unpack_field(xwt_u32):
    hi = lax.bitcast_convert_type(xwt_u32 & _HI, jnp.float32)
    lo = lax.bitcast_convert_type(
        lax.shift_left(xwt_u32, np.uint32(16)), jnp.float32)
    top = jnp.concatenate([hi[WPR * j:WPR * (j + 1), :] for j in range(8)],
                          axis=1)
    bot = jnp.concatenate([lo[WPR * j:WPR * (j + 1), :] for j in range(8)],
                          axis=1)
    return jnp.concatenate([top, bot], axis=0)


def _mlp_body(xw_ref, w1t_ref, b1_ref, w2t_ref, b2_ref, embt_ref, predt_ref):
    xw = xw_ref[...]
    xwt = lax.bitcast_convert_type(xw.T, jnp.uint32)
    for f in range(FPB):
        xt = _unpack_field(xwt[:, QB * f:QB * (f + 1)])
        embt_ref[pl.ds(EMB * f, EMB), :] = xt
        h = jnp.dot(w1t_ref[...], xt, preferred_element_type=jnp.float32)
        h = jnp.maximum(h + b1_ref[...], 0.0)
        p = jnp.dot(w2t_ref[...], h, preferred_element_type=jnp.float32)
        predt_ref[f] = p + b2_ref[...]


def _mlp(emb_c, W1, b1, W2, b2):
    return pl.pallas_call(
        _mlp_body,
        grid=(FIELDS // FPB,),
        in_specs=[
            pl.BlockSpec((FPB * QB, 128), lambda i: (i, 0)),
            pl.BlockSpec((HID, EMB), lambda i: (0, 0)),
            pl.BlockSpec((HID, 1), lambda i: (0, 0)),
            pl.BlockSpec((1, HID), lambda i: (0, 0)),
            pl.BlockSpec((1, 1), lambda i: (0, 0)),
        ],
        out_specs=[
            pl.BlockSpec((FPB * EMB, BATCH), lambda i: (i, 0)),
            pl.BlockSpec((FPB, 1, BATCH), lambda i: (i, 0, 0)),
        ],
        out_shape=[
            jax.ShapeDtypeStruct((FIELDS * EMB, BATCH), jnp.float32),
            jax.ShapeDtypeStruct((FIELDS, 1, BATCH), jnp.float32),
        ],
    )(emb_c, W1.T, b1.reshape(HID, 1), W2.T, b2.reshape(1, 1))


def kernel(id_indices, table, W1, b1, W2, b2):
    idx2d = _sc_shuffle(id_indices.T.astype(jnp.int32)).reshape(IDX_ROWS, IDXV)
    table_c = _pack_table(table.T).reshape(NUM_P, WPR)
    emb2d = _sc_gather(table_c, idx2d)
    emb_c = emb2d.reshape(TOTAL // 8, 128)
    embt, predt = _mlp(emb_c, W1, b1, W2, b2)
    embeddings = embt.reshape(FIELDS, EMB, BATCH).transpose(2, 0, 1)
    prediction = predt.transpose(2, 0, 1)
    return embeddings, prediction


# speedup vs baseline: 10.3803x; 1.0131x over previous
import functools

import jax
import jax.numpy as jnp
import numpy as np
from jax import lax
from jax.experimental import pallas as pl
from jax.experimental.pallas import tpu as pltpu
from jax.experimental.pallas import tpu_sc as plsc

EMB = 32
HID = 16
WPR = EMB // 2
BATCH = 16384
FIELDS = 26
TOTAL = BATCH * FIELDS
NUM_IDS = 1000000
NC = 2
NS = 16
NW = NC * NS
PER_W = TOTAL // NW
IDXV = 128
GPC = 8
CHUNK = IDXV * GPC
NCHUNK = PER_W // CHUNK
IDX_ROWS = TOTAL // IDXV
IDXR_W = PER_W // IDXV

TBLK = 65536
TJ = (NUM_IDS + TBLK - 1) // TBLK
NUM_P = TJ * TBLK
QT = TBLK // 8
QT_SH = QT.bit_length() - 1

_HI = np.uint32(0xFFFF0000)
_RND = np.uint32(0x8000)


def _pack16(a, b):
    au = lax.bitcast_convert_type(a, jnp.uint32) + _RND
    bu = lax.bitcast_convert_type(b, jnp.uint32) + _RND
    return lax.bitcast_convert_type(
        (au & _HI) | lax.shift_right_logical(bu, np.uint32(16)), jnp.int32
    )


def _tr_body(xt_ref, o_ref):
    x = xt_ref[...]
    parts = [
        _pack16(x[0:WPR, QT * j:QT * (j + 1)], x[WPR:EMB, QT * j:QT * (j + 1)])
        for j in range(8)
    ]
    u = jnp.concatenate(parts, axis=0)
    o_ref[...] = u.T


def _pack_table(tableT):
    return pl.pallas_call(
        _tr_body,
        grid=(TJ,),
        in_specs=[pl.BlockSpec((EMB, TBLK), lambda j: (0, j))],
        out_specs=pl.BlockSpec((QT, 128), lambda j: (j, 0)),
        out_shape=jax.ShapeDtypeStruct((NUM_P // 8, 128), jnp.int32),
    )(tableT)


def _perm(i):
    m = i & (TBLK - 1)
    c0 = i - m
    return c0 + 8 * (m & (QT - 1)) + (m >> QT_SH)


def _shuf_body(idxT_hbm, out_hbm, buf, out_v):
    c = lax.axis_index("c")
    s = lax.axis_index("s")
    wid = s * NC + c
    row0 = wid * IDXR_W
    f0 = jnp.minimum(row0 >> 7, FIELDS - 2)
    pltpu.sync_copy(idxT_hbm.at[pl.ds(f0, 2)], buf)
    lane = lax.broadcasted_iota(jnp.int32, (16,), 0)
    base = (lane & 7) * (BATCH // 8) + (lane >> 3)

    def row_body(t, carry):
        rg = row0 + t
        d = (rg >> 7) - f0
        r = rg & 127
        drow = jnp.full((16,), 0, jnp.int32) + d
        for k in range(8):
            v = plsc.load_gather(buf, [drow, base + (16 * r + 2 * k)])
            m = v & (TBLK - 1)
            pv = v - m + 8 * (m & (QT - 1)) + (m >> QT_SH)
            out_v[pl.ds(t * IDXV + 16 * k, 16)] = pv
        return carry

    lax.fori_loop(0, IDXR_W, row_body, 0)
    pltpu.sync_copy(out_v, out_hbm.at[pl.ds(wid * PER_W, PER_W)])


_sc_shuffle = functools.partial(
    pl.kernel,
    mesh=plsc.VectorSubcoreMesh(core_axis_name="c", subcore_axis_name="s"),
    out_type=jax.ShapeDtypeStruct((TOTAL,), jnp.int32),
    scratch_types=[
        pltpu.VMEM((2, BATCH), jnp.int32),
        pltpu.VMEM((PER_W,), jnp.int32),
    ],
    compiler_params=pltpu.CompilerParams(
        use_tc_tiling_on_sc=False, needs_layout_passes=False
    ),
)(_shuf_body)


def _gather_body(table_hbm, idx_hbm, out_hbm, idx_v, rows0, rows1, sem0, sem1):
    c = lax.axis_index("c")
    s = lax.axis_index("s")
    wid = s * NC + c
    base0 = wid * PER_W
    pltpu.sync_copy(idx_hbm.at[pl.ds(wid * IDXR_W, IDXR_W)], idx_v)

    def fire(k, buf, sem):
        for j in range(GPC):
            pltpu.async_copy(
                table_hbm.at[idx_v.at[k * GPC + j]],
                buf.at[pl.ds(j * IDXV, IDXV)],
                sem,
            )

    def drain_write(k, buf, sem):
        pltpu.make_async_copy(out_hbm.at[pl.ds(0, CHUNK)], buf, sem).wait()
        pltpu.sync_copy(buf, out_hbm.at[pl.ds(base0 + k * CHUNK, CHUNK)])

    fire(0, rows0, sem0)

    def pair_body(t, carry):
        k = t * 2
        fire(k + 1, rows1, sem1)
        drain_write(k, rows0, sem0)
        fire(k + 2, rows0, sem0)
        drain_write(k + 1, rows1, sem1)
        return carry

    lax.fori_loop(0, (NCHUNK - 1) // 2, pair_body, 0)
    drain_write(NCHUNK - 1, rows0, sem0)


_sc_gather = functools.partial(
    pl.kernel,
    mesh=plsc.VectorSubcoreMesh(core_axis_name="c", subcore_axis_name="s"),
    out_type=jax.ShapeDtypeStruct((TOTAL, WPR), jnp.int32),
    scratch_types=[
        pltpu.VMEM((IDXR_W, IDXV), jnp.int32),
        pltpu.VMEM((CHUNK, WPR), jnp.int32),
        pltpu.VMEM((CHUNK, WPR), jnp.int32),
        pltpu.SemaphoreType.DMA,
        pltpu.SemaphoreType.DMA,
    ],
    compiler_params=pltpu.CompilerParams(use_tc_tiling_on_sc=False),
)(_gather_body)


QB = BATCH // 8


FPB = 2


def _unpack_field(xwt_u32):
    hi = lax.bitcast_convert_type(xwt_u32 & _HI, jnp.float32)
    lo = lax.bitcast_convert_type(
        lax.shift_left(xwt_u32, np.uint32(16)), jnp.float32)
    top = jnp.concatenate([hi[WPR * j:WPR * (j + 1), :] for j in range(8)],
                          axis=1)
    bot = jnp.concatenate([lo[WPR * j:WPR * (j + 1), :] for j in range(8)],
                          axis=1)
    return jnp.concatenate([top, bot], axis=0)


def _mlp_body(xw_ref, w1t_ref, b1_ref, w2t_ref, b2_ref, embt_ref, predt_ref):
    xw = xw_ref[...]
    xwt = lax.bitcast_convert_type(xw.T, jnp.uint32)
    for f in range(FPB):
        xt = _unpack_field(xwt[:, QB * f:QB * (f + 1)])
        embt_ref[pl.ds(EMB * f, EMB), :] = xt
        h = jnp.dot(w1t_ref[...], xt, preferred_element_type=jnp.float32)
        h = jnp.maximum(h + b1_ref[...], 0.0)
        p = jnp.dot(w2t_ref[...], h, preferred_element_type=jnp.float32)
        predt_ref[f] = p + b2_ref[...]


def _mlp(emb_c, W1, b1, W2, b2):
    return pl.pallas_call(
        _mlp_body,
        grid=(FIELDS // FPB,),
        in_specs=[
            pl.BlockSpec((FPB * QB, 128), lambda i: (i, 0)),
            pl.BlockSpec((HID, EMB), lambda i: (0, 0)),
            pl.BlockSpec((HID, 1), lambda i: (0, 0)),
            pl.BlockSpec((1, HID), lambda i: (0, 0)),
            pl.BlockSpec((1, 1), lambda i: (0, 0)),
        ],
        out_specs=[
            pl.BlockSpec((FPB * EMB, BATCH), lambda i: (i, 0)),
            pl.BlockSpec((FPB, 1, BATCH), lambda i: (i, 0, 0)),
        ],
        out_shape=[
            jax.ShapeDtypeStruct((FIELDS * EMB, BATCH), jnp.float32),
            jax.ShapeDtypeStruct((FIELDS, 1, BATCH), jnp.float32),
        ],
    )(emb_c, W1.T, b1.reshape(HID, 1), W2.T, b2.reshape(1, 1))


def kernel(id_indices, table, W1, b1, W2, b2):
    idx2d = _sc_shuffle(id_indices.T.astype(jnp.int32)).reshape(IDX_ROWS, IDXV)
    table_c = _pack_table(table.T).reshape(NUM_P, WPR)
    emb2d = _sc_gather(table_c, idx2d)
    emb_c = emb2d.reshape(TOTAL // 8, 128)
    embt, predt = _mlp(emb_c, W1, b1, W2, b2)
    embeddings = embt.reshape(FIELDS, EMB, BATCH).transpose(2, 0, 1)
    prediction = predt.transpose(2, 0, 1)
    return embeddings, prediction
